# baseline (speedup 1.0000x reference)
import functools

import jax
import jax.numpy as jnp
from jax import lax
from jax.experimental import pallas as pl
from jax.experimental.pallas import tpu as pltpu
from jax.experimental.pallas import tpu_sc as plsc

NUM_USERS = 100000
NUM_ITEMS = 100000
EMBED_DIM = 64
BATCH = 4096

_INFO = plsc.get_sparse_core_info()
_NC, _NS, _L = _INFO.num_cores, _INFO.num_subcores, _INFO.num_lanes
_NW = _NC * _NS
_BPW = BATCH // _NW
_GROUPS = _BPW // _L


def _stage1(user_hbm, uid_hbm, iid_hbm, ua_out, ia_out,
            uid_v, iid_v, gc_v, rows_ua, rows_ia, sem):
    wid = lax.axis_index("s") * _NC + lax.axis_index("c")
    base = wid * _BPW

    pltpu.sync_copy(uid_hbm.at[pl.ds(base, _BPW)], uid_v.at[pl.ds(0, _BPW)])
    pltpu.sync_copy(iid_hbm.at[pl.ds(base, _BPW)], iid_v.at[pl.ds(0, _BPW)])

    for g in range(_GROUPS):
        sl = pl.ds(g * _L, _L)
        i = iid_v[sl]
        si = jnp.logical_and(
            jnp.logical_and(jnp.equal(jnp.remainder(i - 3, 1500), 0), i >= 3),
            i <= 94503)
        gc_v[sl] = jnp.where(si, i - 3, i)

    def fire(b, _):
        ga = uid_v[pl.ds(b, _L)][0]
        gc = gc_v[pl.ds(b, _L)][0]
        pltpu.async_copy(user_hbm.at[pl.ds(ga, 1)],
                         rows_ua.at[pl.ds(b, 1)], sem)
        pltpu.async_copy(user_hbm.at[pl.ds(gc, 1)],
                         rows_ia.at[pl.ds(b, 1)], sem)
        return ()

    lax.fori_loop(0, _BPW, fire, ())
    for buf in (rows_ua, rows_ia):
        pltpu.make_async_copy(user_hbm.at[pl.ds(0, _BPW)], buf, sem).wait()

    pltpu.sync_copy(rows_ua, ua_out.at[pl.ds(base, _BPW)])
    pltpu.sync_copy(rows_ia, ia_out.at[pl.ds(base, _BPW)])


def _stage2(item_hbm, uid_hbm, iid_hbm, ua_hbm, ia_hbm, out_hbm,
            uid_v, iid_v, gb_v,
            cu1_v, cu2_v, ci1_v, ci2_v,
            rows_ua, rows_ub, rows_ia, rows_ib, out_v, sem):
    wid = lax.axis_index("s") * _NC + lax.axis_index("c")
    base = wid * _BPW

    pltpu.sync_copy(uid_hbm.at[pl.ds(base, _BPW)], uid_v.at[pl.ds(0, _BPW)])
    pltpu.sync_copy(iid_hbm.at[pl.ds(base, _BPW)], iid_v.at[pl.ds(0, _BPW)])
    pltpu.sync_copy(ua_hbm.at[pl.ds(base, _BPW)], rows_ua)
    pltpu.sync_copy(ia_hbm.at[pl.ds(base, _BPW)], rows_ia)

    for g in range(_GROUPS):
        sl = pl.ds(g * _L, _L)
        u = uid_v[sl]
        su = jnp.logical_and(jnp.equal(jnp.remainder(u, 1500), 0),
                             u <= 94500)
        gb_v[sl] = jnp.where(su, u + 3, u)
        half = jnp.full((_L,), 0.5, jnp.float32)
        quarter = jnp.full((_L,), 0.25, jnp.float32)
        zero = jnp.zeros((_L,), jnp.float32)
        cu1_v[sl] = jnp.where(su, half, quarter)
        cu2_v[sl] = jnp.where(su, half, zero)

        i = iid_v[sl]
        si = jnp.logical_and(
            jnp.logical_and(jnp.equal(jnp.remainder(i - 3, 1500), 0), i >= 3),
            i <= 94503)
        ci1_v[sl] = jnp.where(si, half, zero)
        ci2_v[sl] = jnp.where(si, half, quarter)

    def fire(b, _):
        gb = gb_v[pl.ds(b, _L)][0]
        gd = iid_v[pl.ds(b, _L)][0]
        pltpu.async_copy(item_hbm.at[pl.ds(gb, 1)],
                         rows_ub.at[pl.ds(b, 1)], sem)
        pltpu.async_copy(item_hbm.at[pl.ds(gd, 1)],
                         rows_ib.at[pl.ds(b, 1)], sem)
        return ()

    lax.fori_loop(0, _BPW, fire, ())
    for buf in (rows_ub, rows_ib):
        pltpu.make_async_copy(item_hbm.at[pl.ds(0, _BPW)], buf, sem).wait()

    lane = lax.iota(jnp.int32, _L)
    for g in range(_GROUPS):
        sl = pl.ds(g * _L, _L)
        lrow = lane + g * _L
        cu1 = cu1_v[sl]
        cu2 = cu2_v[sl]
        ci1 = ci1_v[sl]
        ci2 = ci2_v[sl]

        def body(d, acc):
            col = jnp.bitwise_and(lane + d, EMBED_DIM - 1)
            ua = plsc.load_gather(rows_ua, [lrow, col])
            ub = plsc.load_gather(rows_ub, [lrow, col])
            ia = plsc.load_gather(rows_ia, [lrow, col])
            ib = plsc.load_gather(rows_ib, [lrow, col])
            ue = cu1 * ua + cu2 * ub
            ie = ci1 * ia + ci2 * ib
            return acc + ue * ie

        out_v[sl] = lax.fori_loop(0, EMBED_DIM, body,
                                  jnp.zeros((_L,), jnp.float32))

    pltpu.sync_copy(out_v, out_hbm.at[pl.ds(base, _BPW)])


@jax.jit
def _run(user_emb, item_emb, user_ids, item_ids):
    mesh = plsc.VectorSubcoreMesh(core_axis_name="c", subcore_axis_name="s")
    params = pltpu.CompilerParams(
        needs_layout_passes=False, use_tc_tiling_on_sc=True)

    k1 = functools.partial(
        pl.kernel,
        mesh=mesh,
        compiler_params=params,
        out_type=[
            jax.ShapeDtypeStruct((BATCH, EMBED_DIM), jnp.float32),
            jax.ShapeDtypeStruct((BATCH, EMBED_DIM), jnp.float32),
        ],
        scratch_types=[
            pltpu.VMEM((_BPW + _L,), jnp.int32),
            pltpu.VMEM((_BPW + _L,), jnp.int32),
            pltpu.VMEM((_BPW + _L,), jnp.int32),
            pltpu.VMEM((_BPW, EMBED_DIM), jnp.float32),
            pltpu.VMEM((_BPW, EMBED_DIM), jnp.float32),
            pltpu.SemaphoreType.DMA,
        ],
    )(_stage1)

    k2 = functools.partial(
        pl.kernel,
        mesh=mesh,
        compiler_params=params,
        out_type=jax.ShapeDtypeStruct((BATCH,), jnp.float32),
        scratch_types=[
            pltpu.VMEM((_BPW + _L,), jnp.int32),
            pltpu.VMEM((_BPW + _L,), jnp.int32),
            pltpu.VMEM((_BPW + _L,), jnp.int32),
            pltpu.VMEM((_BPW,), jnp.float32),
            pltpu.VMEM((_BPW,), jnp.float32),
            pltpu.VMEM((_BPW,), jnp.float32),
            pltpu.VMEM((_BPW,), jnp.float32),
            pltpu.VMEM((_BPW, EMBED_DIM), jnp.float32),
            pltpu.VMEM((_BPW, EMBED_DIM), jnp.float32),
            pltpu.VMEM((_BPW, EMBED_DIM), jnp.float32),
            pltpu.VMEM((_BPW, EMBED_DIM), jnp.float32),
            pltpu.VMEM((_BPW,), jnp.float32),
            pltpu.SemaphoreType.DMA,
        ],
    )(_stage2)

    ua_rows, ia_rows = k1(user_emb, user_ids, item_ids)
    return k2(item_emb, user_ids, item_ids, ua_rows, ia_rows)


def kernel(user_emb, item_emb, user_ids, item_ids):
    return _run(user_emb, item_emb,
                user_ids.astype(jnp.int32), item_ids.astype(jnp.int32))


# speedup vs baseline: 1.0343x; 1.0343x over previous
import functools

import jax
import jax.numpy as jnp
from jax import lax
from jax.experimental import pallas as pl
from jax.experimental.pallas import tpu as pltpu
from jax.experimental.pallas import tpu_sc as plsc

NUM_USERS = 100000
NUM_ITEMS = 100000
EMBED_DIM = 64
BATCH = 4096

_INFO = plsc.get_sparse_core_info()
_NC, _NS, _L = _INFO.num_cores, _INFO.num_subcores, _INFO.num_lanes
_NW = _NC * _NS
_BPW = BATCH // _NW
_GROUPS = _BPW // _L


def _sc_kernel(user_hbm, item_hbm, uid_hbm, iid_hbm, out_hbm,
               uid_v, iid_v, gb_v, gc_v,
               cu1_v, cu2_v, ci1_v, ci2_v,
               rows_ua, rows_ub, rows_ia, rows_ib, out_v, sem):
    wid = lax.axis_index("s") * _NC + lax.axis_index("c")
    base = wid * _BPW

    pltpu.sync_copy(uid_hbm.at[pl.ds(base, _BPW)], uid_v.at[pl.ds(0, _BPW)])
    pltpu.sync_copy(iid_hbm.at[pl.ds(base, _BPW)], iid_v.at[pl.ds(0, _BPW)])

    for g in range(_GROUPS):
        sl = pl.ds(g * _L, _L)
        u = uid_v[sl]
        su = jnp.logical_and(jnp.equal(jnp.remainder(u, 1500), 0),
                             u <= 94500)
        gb_v[sl] = jnp.where(su, u + 3, u)
        half = jnp.full((_L,), 0.5, jnp.float32)
        quarter = jnp.full((_L,), 0.25, jnp.float32)
        zero = jnp.zeros((_L,), jnp.float32)
        cu1_v[sl] = jnp.where(su, half, quarter)
        cu2_v[sl] = jnp.where(su, half, zero)

        i = iid_v[sl]
        si = jnp.logical_and(
            jnp.logical_and(jnp.equal(jnp.remainder(i - 3, 1500), 0), i >= 3),
            i <= 94503)
        gc_v[sl] = jnp.where(si, i - 3, i)
        ci1_v[sl] = jnp.where(si, half, zero)
        ci2_v[sl] = jnp.where(si, half, quarter)

    def fire(b, _):
        ga = uid_v[pl.ds(b, _L)][0]
        gb = gb_v[pl.ds(b, _L)][0]
        gc = gc_v[pl.ds(b, _L)][0]
        gd = iid_v[pl.ds(b, _L)][0]
        pltpu.async_copy(user_hbm.at[pl.ds(ga, 1)],
                         rows_ua.at[pl.ds(b, 1)], sem)
        pltpu.async_copy(item_hbm.at[pl.ds(gb, 1)],
                         rows_ub.at[pl.ds(b, 1)], sem)
        pltpu.async_copy(user_hbm.at[pl.ds(gc, 1)],
                         rows_ia.at[pl.ds(b, 1)], sem)
        pltpu.async_copy(item_hbm.at[pl.ds(gd, 1)],
                         rows_ib.at[pl.ds(b, 1)], sem)
        return ()

    lax.fori_loop(0, _BPW, fire, ())
    for buf in (rows_ua, rows_ub, rows_ia, rows_ib):
        pltpu.make_async_copy(user_hbm.at[pl.ds(0, _BPW)], buf, sem).wait()

    lane = lax.iota(jnp.int32, _L)
    for g in range(_GROUPS):
        sl = pl.ds(g * _L, _L)
        lrow = lane + g * _L
        cu1 = cu1_v[sl]
        cu2 = cu2_v[sl]
        ci1 = ci1_v[sl]
        ci2 = ci2_v[sl]

        def body(d, acc):
            col = jnp.bitwise_and(lane + d, EMBED_DIM - 1)
            ua = plsc.load_gather(rows_ua, [lrow, col])
            ub = plsc.load_gather(rows_ub, [lrow, col])
            ia = plsc.load_gather(rows_ia, [lrow, col])
            ib = plsc.load_gather(rows_ib, [lrow, col])
            ue = cu1 * ua + cu2 * ub
            ie = ci1 * ia + ci2 * ib
            return acc + ue * ie

        out_v[sl] = lax.fori_loop(0, EMBED_DIM, body,
                                  jnp.zeros((_L,), jnp.float32))

    pltpu.sync_copy(out_v, out_hbm.at[pl.ds(base, _BPW)])


@jax.jit
def _run(user_emb, item_emb, user_ids, item_ids):
    mesh = plsc.VectorSubcoreMesh(core_axis_name="c", subcore_axis_name="s")
    kern = functools.partial(
        pl.kernel,
        mesh=mesh,
        compiler_params=pltpu.CompilerParams(
            needs_layout_passes=False, use_tc_tiling_on_sc=True),
        out_type=jax.ShapeDtypeStruct((BATCH,), jnp.float32),
        scratch_types=[
            pltpu.VMEM((_BPW + _L,), jnp.int32),
            pltpu.VMEM((_BPW + _L,), jnp.int32),
            pltpu.VMEM((_BPW + _L,), jnp.int32),
            pltpu.VMEM((_BPW + _L,), jnp.int32),
            pltpu.VMEM((_BPW,), jnp.float32),
            pltpu.VMEM((_BPW,), jnp.float32),
            pltpu.VMEM((_BPW,), jnp.float32),
            pltpu.VMEM((_BPW,), jnp.float32),
            pltpu.VMEM((_BPW, EMBED_DIM), jnp.float32),
            pltpu.VMEM((_BPW, EMBED_DIM), jnp.float32),
            pltpu.VMEM((_BPW, EMBED_DIM), jnp.float32),
            pltpu.VMEM((_BPW, EMBED_DIM), jnp.float32),
            pltpu.VMEM((_BPW,), jnp.float32),
            pltpu.SemaphoreType.DMA,
        ],
    )(_sc_kernel)
    return kern(user_emb, item_emb, user_ids, item_ids)


def kernel(user_emb, item_emb, user_ids, item_ids):
    return _run(user_emb, item_emb,
                user_ids.astype(jnp.int32), item_ids.astype(jnp.int32))
